# baseline (speedup 1.0000x reference)
import functools
import math

import jax
import jax.numpy as jnp
from jax import lax
from jax.experimental import pallas as pl
from jax.experimental.pallas import tpu as pltpu
from jax.experimental.pallas import tpu_sc as plsc

N = 10000
E = 160000
D = 256
SCALE = math.sqrt(D)
NPAD = 10240
ROWB = 512

NC = 2
NS = 16
LANES = 16
NW = NC * NS
EPAD = 163840
EW = EPAD // NW

@functools.cache
def _sc_mesh():
    return plsc.VectorSubcoreMesh(core_axis_name="c", subcore_axis_name="s",
                                  num_cores=NC, num_subcores=NS)


@functools.cache
def _sc_params():
    import dataclasses
    cp = pltpu.CompilerParams()
    if "needs_layout_passes" in pltpu.CompilerParams.__dataclass_fields__:
        cp = dataclasses.replace(cp, needs_layout_passes=False)
    return cp


def _worker_id():
    return lax.axis_index("c") * NS + lax.axis_index("s")


def _permute(v, p):
    dnums = lax.GatherDimensionNumbers(
        offset_dims=(), collapsed_slice_dims=(0,), start_index_map=(0,))
    return lax.gather(v, p[:, None], dnums, slice_sizes=(1,),
                      mode=lax.GatherScatterMode.PROMISE_IN_BOUNDS)


def _sca_body(px_hbm, py_hbm, pz_hbm, src_hbm, dst_hbm,
              rpx_hbm, rpy_hbm, rpz_hbm,
              px_v, py_v, pz_v, si_v, di_v, rx_v, ry_v, rz_v, sem):
    base = _worker_id() * EW
    pltpu.sync_copy(px_hbm, px_v)
    pltpu.sync_copy(py_hbm, py_v)
    pltpu.sync_copy(pz_hbm, pz_v)
    pltpu.async_copy(src_hbm.at[pl.ds(base, EW)], si_v, sem).wait()
    pltpu.async_copy(dst_hbm.at[pl.ds(base, EW)], di_v, sem).wait()

    @pl.loop(0, EW, step=LANES)
    def _(o):
        s16 = si_v[pl.ds(o, LANES)]
        d16 = di_v[pl.ds(o, LANES)]
        rx_v[pl.ds(o, LANES)] = (plsc.load_gather(px_v, [s16])
                                 - plsc.load_gather(px_v, [d16]))
        ry_v[pl.ds(o, LANES)] = (plsc.load_gather(py_v, [s16])
                                 - plsc.load_gather(py_v, [d16]))
        rz_v[pl.ds(o, LANES)] = (plsc.load_gather(pz_v, [s16])
                                 - plsc.load_gather(pz_v, [d16]))

    pltpu.sync_copy(rx_v, rpx_hbm.at[pl.ds(base, EW)])
    pltpu.sync_copy(ry_v, rpy_hbm.at[pl.ds(base, EW)])
    pltpu.sync_copy(rz_v, rpz_hbm.at[pl.ds(base, EW)])


def _sc_relpos(posx, posy, posz, srcp, dstp):
    f32 = jnp.float32
    kern = pl.kernel(
        _sca_body,
        out_type=[jax.ShapeDtypeStruct((EPAD,), f32)] * 3,
        mesh=_sc_mesh(),
        compiler_params=_sc_params(),
        scratch_types=[pltpu.VMEM((N,), f32)] * 3
        + [pltpu.VMEM((EW,), jnp.int32)] * 2
        + [pltpu.VMEM((EW,), f32)] * 3
        + [pltpu.SemaphoreType.DMA],
    )
    return kern(posx, posy, posz, srcp, dstp)


GB = 256
EPB = EPAD // NS


def _scb_body(q0_hbm, k0_hbm, q1_hbm, k1_hbm, src_hbm, dst_hbm,
              p0_hbm, p1_hbm,
              si_v, di_v, qb_v, kb_v, pb_v, sem):
    cid = lax.axis_index("c")
    sid = lax.axis_index("s")
    base = sid * EPB
    lanes = lax.iota(jnp.int32, LANES)
    perms = [(lanes + k) & (LANES - 1) for k in (8, 4, 2, 1)]
    lane0 = lanes == 0

    def do_half(q_hbm, k_hbm, p_hbm):
        @pl.loop(0, EPB, step=GB)
        def _(b):
            eb = base + b
            pltpu.async_copy(src_hbm.at[pl.ds(eb, GB)], si_v, sem).wait()
            pltpu.async_copy(dst_hbm.at[pl.ds(eb, GB)], di_v, sem).wait()
            pltpu.async_copy(q_hbm.at[di_v], qb_v, sem).wait()
            pltpu.async_copy(k_hbm.at[si_v], kb_v, sem).wait()

            @pl.loop(0, GB, step=LANES)
            def _(g):
                for e in range(LANES):
                    row = g + e
                    acc0 = (qb_v[row, pl.ds(0, LANES)]
                            * kb_v[row, pl.ds(0, LANES)])
                    acc1 = (qb_v[row, pl.ds(LANES, LANES)]
                            * kb_v[row, pl.ds(LANES, LANES)])
                    for c in range(2, 8, 2):
                        acc0 += (qb_v[row, pl.ds(c * LANES, LANES)]
                                 * kb_v[row, pl.ds(c * LANES, LANES)])
                        acc1 += (qb_v[row, pl.ds((c + 1) * LANES, LANES)]
                                 * kb_v[row, pl.ds((c + 1) * LANES, LANES)])
                    r = acc0 + acc1
                    for p in perms:
                        r = r + _permute(r, p)
                    plsc.store_scatter(
                        pb_v, [jnp.full((LANES,), row, jnp.int32)], r,
                        mask=lane0)

            pltpu.sync_copy(pb_v, p_hbm.at[pl.ds(eb, GB)])

    @pl.when(cid == 0)
    def _():
        do_half(q0_hbm, k0_hbm, p0_hbm)

    @pl.when(cid == 1)
    def _():
        do_half(q1_hbm, k1_hbm, p1_hbm)


def _sc_partial_dots(q0, k0, q1, k1, srcp, dstp):
    f32 = jnp.float32
    kern = pl.kernel(
        _scb_body,
        out_type=[jax.ShapeDtypeStruct((EPAD,), f32)] * 2,
        mesh=_sc_mesh(),
        compiler_params=_sc_params(),
        scratch_types=[pltpu.VMEM((GB,), jnp.int32)] * 2
        + [pltpu.VMEM((GB, 128), f32)] * 2
        + [pltpu.VMEM((GB,), f32)]
        + [pltpu.SemaphoreType.DMA],
    )
    return kern(q0, k0, q1, k1, srcp, dstp)


EB = 640
NEB = EPAD // EB


def _posmlp_body(rpx_ref, rpy_ref, rpz_ref, cols_ref, wp2t_ref,
                 pe0t_ref, pe1t_ref):
    rx = rpx_ref[0]
    ry = rpy_ref[0]
    rz = rpz_ref[0]
    cols = cols_ref[...]
    p = jnp.maximum(cols[:, 0:1] * rx + cols[:, 1:2] * ry
                    + cols[:, 2:3] * rz + cols[:, 3:4], 0.0)
    pet = (jnp.dot(wp2t_ref[...], p, preferred_element_type=jnp.float32,
                   precision=jax.lax.Precision.HIGHEST)
           + cols[:, 4:5])
    pe0t_ref[...] = pet[:128]
    pe1t_ref[...] = pet[128:]


def _tc_pos_mlp(rpx, rpy, rpz, Wp1, bp1, Wp2, bp2):
    f32 = jnp.float32
    rpx3 = rpx.reshape(NEB, 1, EB)
    rpy3 = rpy.reshape(NEB, 1, EB)
    rpz3 = rpz.reshape(NEB, 1, EB)
    cols = jnp.concatenate(
        [Wp1.T, bp1[:, None], bp2[:, None]], axis=1)
    wp2t = Wp2.T
    r_spec = pl.BlockSpec((1, 1, EB), lambda i: (i, 0, 0))
    out = pl.pallas_call(
        _posmlp_body,
        grid=(NEB,),
        in_specs=[r_spec, r_spec, r_spec,
                  pl.BlockSpec((D, 5), lambda i: (0, 0)),
                  pl.BlockSpec((D, D), lambda i: (0, 0))],
        out_specs=[pl.BlockSpec((128, EB), lambda i: (0, i))] * 2,
        out_shape=[jax.ShapeDtypeStruct((128, EPAD), f32)] * 2,
    )(rpx3, rpy3, rpz3, cols, wp2t)
    return out


GBC = 128
ROWS_PER_TILE = NPAD // NS


def _scc_body(v0_hbm, v1_hbm, pe0_hbm, pe1_hbm, p0_hbm, p1_hbm,
              src_hbm, dst_hbm, zrows_hbm, zvec_hbm,
              s0_hbm, s1_hbm, den_hbm,
              si_v, di_v, vb_v, peb_v, p0b_v, p1b_v, wb_v, acc_sh, dacc_sh,
              sem):
    cid = lax.axis_index("c")
    sid = lax.axis_index("s")
    base = sid * EPB
    zbase = sid * ROWS_PER_TILE
    lanes = lax.iota(jnp.int32, LANES)
    chunk_rows = [lanes + c * LANES for c in range(8)]
    inv_scale = jnp.float32(1.0 / SCALE)

    pltpu.sync_copy(zrows_hbm.at[pl.ds(zbase, ROWS_PER_TILE)],
                    acc_sh.at[pl.ds(zbase, ROWS_PER_TILE)])

    @pl.when(cid == 0)
    def _():
        pltpu.sync_copy(zvec_hbm.at[pl.ds(zbase, ROWS_PER_TILE)],
                        dacc_sh.at[pl.ds(zbase, ROWS_PER_TILE)])

    plsc.subcore_barrier()

    def do_half(v_hbm, pe_hbm):
        @pl.loop(0, EPB, step=GBC)
        def _(b):
            eb = base + b
            pltpu.async_copy(src_hbm.at[pl.ds(eb, GBC)], si_v, sem).wait()
            pltpu.async_copy(dst_hbm.at[pl.ds(eb, GBC)], di_v, sem).wait()
            pltpu.async_copy(v_hbm.at[si_v], vb_v, sem).wait()
            pltpu.async_copy(pe_hbm.at[:, pl.ds(eb, GBC)], peb_v, sem).wait()
            pltpu.async_copy(p0_hbm.at[pl.ds(eb, GBC)], p0b_v, sem).wait()
            pltpu.async_copy(p1_hbm.at[pl.ds(eb, GBC)], p1b_v, sem).wait()

            @pl.loop(0, GBC, step=LANES)
            def _(g):
                w16 = jnp.exp((p0b_v[pl.ds(g, LANES)]
                               + p1b_v[pl.ds(g, LANES)]) * inv_scale)
                eid = eb + g + lanes
                w16 = jnp.where(eid < E, w16, jnp.float32(0.0))
                wb_v[pl.ds(g, LANES)] = w16
                for e in range(LANES):
                    row = g + e
                    rowsplat = jnp.full((LANES,), row, jnp.int32)
                    wvec = plsc.load_gather(wb_v, [rowsplat])
                    for c in range(8):
                        sl = pl.ds(c * LANES, LANES)
                        pec = plsc.load_gather(
                            peb_v, [chunk_rows[c], rowsplat])
                        vb_v[row, sl] = (vb_v[row, sl] + pec) * wvec

            pltpu.sync_copy(vb_v, acc_sh.at[di_v], add=True)

            @pl.when(cid == 0)
            def _():
                pltpu.sync_copy(wb_v, dacc_sh.at[di_v], add=True)

    @pl.when(cid == 0)
    def _():
        do_half(v0_hbm, pe0_hbm)

    @pl.when(cid == 1)
    def _():
        do_half(v1_hbm, pe1_hbm)

    plsc.subcore_barrier()

    @pl.when(cid == 0)
    def _():
        pltpu.sync_copy(acc_sh.at[pl.ds(zbase, ROWS_PER_TILE)],
                        s0_hbm.at[pl.ds(zbase, ROWS_PER_TILE)])
        pltpu.sync_copy(dacc_sh.at[pl.ds(zbase, ROWS_PER_TILE)],
                        den_hbm.at[pl.ds(zbase, ROWS_PER_TILE)])

    @pl.when(cid == 1)
    def _():
        pltpu.sync_copy(acc_sh.at[pl.ds(zbase, ROWS_PER_TILE)],
                        s1_hbm.at[pl.ds(zbase, ROWS_PER_TILE)])


def _sc_aggregate(v0, v1, pe0, pe1, p0, p1, srcp, dstp):
    f32 = jnp.float32
    zrows = jnp.zeros((NPAD, 128), f32)
    zvec = jnp.zeros((NPAD,), f32)
    kern = pl.kernel(
        _scc_body,
        out_type=[jax.ShapeDtypeStruct((NPAD, 128), f32)] * 2
        + [jax.ShapeDtypeStruct((NPAD,), f32)],
        mesh=_sc_mesh(),
        compiler_params=_sc_params(),
        scratch_types=[pltpu.VMEM((GBC,), jnp.int32)] * 2
        + [pltpu.VMEM((GBC, 128), f32)] * 2
        + [pltpu.VMEM((GBC,), f32)] * 3
        + [pltpu.VMEM_SHARED((NPAD, 128), f32),
           pltpu.VMEM_SHARED((NPAD,), f32),
           pltpu.SemaphoreType.DMA],
    )
    return kern(v0, v1, pe0, pe1, p0, p1, srcp, dstp, zrows, zvec)


def _dense_body(x_ref, w1, b1, w2, b2, wq, bq, wk, bk, wv, bv,
                h_ref, q0_ref, q1_ref, k0_ref, k1_ref, v0_ref, v1_ref):
    x = x_ref[...]
    h1 = jnp.maximum(jnp.dot(x, w1[...], preferred_element_type=jnp.float32)
                     + b1[...], 0.0)
    h = jnp.dot(h1, w2[...], preferred_element_type=jnp.float32) + b2[...]
    h_ref[...] = h
    q = jnp.dot(h, wq[...], preferred_element_type=jnp.float32) + bq[...]
    k = jnp.dot(h, wk[...], preferred_element_type=jnp.float32) + bk[...]
    v = jnp.dot(h, wv[...], preferred_element_type=jnp.float32) + bv[...]
    q0_ref[...] = q[:, :128]
    q1_ref[...] = q[:, 128:]
    k0_ref[...] = k[:, :128]
    k1_ref[...] = k[:, 128:]
    v0_ref[...] = v[:, :128]
    v1_ref[...] = v[:, 128:]


def _dense_qkv(xp, W1, b1, W2, b2, Wq, bq, Wk, bk, Wv, bv):
    grid = (NPAD // ROWB,)
    row_spec = pl.BlockSpec((ROWB, D), lambda i: (i, 0))
    half_spec = pl.BlockSpec((ROWB, 128), lambda i: (i, 0))
    w_spec = pl.BlockSpec((D, D), lambda i: (0, 0))
    b_spec = pl.BlockSpec((1, D), lambda i: (0, 0))
    out = pl.pallas_call(
        _dense_body,
        grid=grid,
        in_specs=[row_spec] + [w_spec, b_spec] * 5,
        out_specs=[row_spec] + [half_spec] * 6,
        out_shape=[jax.ShapeDtypeStruct((NPAD, D), jnp.float32)]
        + [jax.ShapeDtypeStruct((NPAD, 128), jnp.float32)] * 6,
    )(xp, W1, b1.reshape(1, D), W2, b2.reshape(1, D),
      Wq, bq.reshape(1, D), Wk, bk.reshape(1, D), Wv, bv.reshape(1, D))
    return out


def _finish_body(s0_ref, s1_ref, den_ref, h_ref, x_ref, g1, be1, g2, be2, o_ref):
    s = jnp.concatenate([s0_ref[...], s1_ref[...]], axis=-1)
    den = den_ref[...]
    h = h_ref[...]
    x = x_ref[...]
    out = s / (den + 1e-16)
    a = out + h
    mu = jnp.mean(a, axis=-1, keepdims=True)
    var = jnp.mean((a - mu) ** 2, axis=-1, keepdims=True)
    h_attn = (a - mu) * jax.lax.rsqrt(var + 1e-5) * g1[...] + be1[...]
    b = h_attn + x
    mu2 = jnp.mean(b, axis=-1, keepdims=True)
    var2 = jnp.mean((b - mu2) ** 2, axis=-1, keepdims=True)
    o_ref[...] = (b - mu2) * jax.lax.rsqrt(var2 + 1e-5) * g2[...] + be2[...]


def _finish(s0, s1, den, hp, xp, g1, be1, g2, be2):
    grid = (NPAD // ROWB,)
    row_spec = pl.BlockSpec((ROWB, D), lambda i: (i, 0))
    half_spec = pl.BlockSpec((ROWB, 128), lambda i: (i, 0))
    den_spec = pl.BlockSpec((ROWB, 1), lambda i: (i, 0))
    v_spec = pl.BlockSpec((1, D), lambda i: (0, 0))
    return pl.pallas_call(
        _finish_body,
        grid=grid,
        in_specs=[half_spec, half_spec, den_spec, row_spec, row_spec]
        + [v_spec] * 4,
        out_specs=row_spec,
        out_shape=jax.ShapeDtypeStruct((NPAD, D), jnp.float32),
    )(s0, s1, den.reshape(NPAD, 1), hp, xp,
      g1.reshape(1, D), be1.reshape(1, D), g2.reshape(1, D), be2.reshape(1, D))


def kernel(x, edge_index, pos, W1, b1, W2, b2, Wq, bq, Wk, bk, Wv, bv,
           Wp1, bp1, Wp2, bp2, g1, be1, g2, be2):
    xp = jnp.pad(x, ((0, NPAD - N), (0, 0)))
    hp, q0, q1, k0, k1, v0, v1 = _dense_qkv(
        xp, W1, b1, W2, b2, Wq, bq, Wk, bk, Wv, bv)

    src = edge_index[0]
    dst = edge_index[1]
    srcp = jnp.pad(src, (0, EPAD - E))
    dstp = jnp.pad(dst, (0, EPAD - E))
    posx = pos[:, 0]
    posy = pos[:, 1]
    posz = pos[:, 2]

    rpx, rpy, rpz = _sc_relpos(posx, posy, posz, srcp, dstp)

    p0, p1 = _sc_partial_dots(q0, k0, q1, k1, srcp, dstp)

    pe0t, pe1t = _tc_pos_mlp(rpx, rpy, rpz, Wp1, bp1, Wp2, bp2)
    s0, s1, den = _sc_aggregate(v0, v1, pe0t, pe1t, p0, p1, srcp, dstp)
    out = _finish(s0, s1, den, hp, xp, g1, be1, g2, be2)
    return out[:N]


# speedup vs baseline: 2.7413x; 2.7413x over previous
import functools
import math

import jax
import jax.numpy as jnp
from jax import lax
from jax.experimental import pallas as pl
from jax.experimental.pallas import tpu as pltpu
from jax.experimental.pallas import tpu_sc as plsc

N = 10000
E = 160000
D = 256
SCALE = math.sqrt(D)
NPAD = 10240
ROWB = 512

NC = 2
NS = 16
LANES = 16
NW = NC * NS
EPAD = 163840
EW = EPAD // NW

@functools.cache
def _sc_mesh():
    return plsc.VectorSubcoreMesh(core_axis_name="c", subcore_axis_name="s",
                                  num_cores=NC, num_subcores=NS)


@functools.cache
def _sc_params():
    import dataclasses
    cp = pltpu.CompilerParams()
    if "needs_layout_passes" in pltpu.CompilerParams.__dataclass_fields__:
        cp = dataclasses.replace(cp, needs_layout_passes=False)
    return cp


def _worker_id():
    return lax.axis_index("c") * NS + lax.axis_index("s")


def _permute(v, p):
    dnums = lax.GatherDimensionNumbers(
        offset_dims=(), collapsed_slice_dims=(0,), start_index_map=(0,))
    return lax.gather(v, p[:, None], dnums, slice_sizes=(1,),
                      mode=lax.GatherScatterMode.PROMISE_IN_BOUNDS)


def _sca_body(px_hbm, py_hbm, pz_hbm, src_hbm, dst_hbm,
              rpx_hbm, rpy_hbm, rpz_hbm,
              px_v, py_v, pz_v, si_v, di_v, rx_v, ry_v, rz_v, sem):
    base = _worker_id() * EW
    pltpu.sync_copy(px_hbm, px_v)
    pltpu.sync_copy(py_hbm, py_v)
    pltpu.sync_copy(pz_hbm, pz_v)
    pltpu.async_copy(src_hbm.at[pl.ds(base, EW)], si_v, sem).wait()
    pltpu.async_copy(dst_hbm.at[pl.ds(base, EW)], di_v, sem).wait()

    @pl.loop(0, EW, step=LANES)
    def _(o):
        s16 = si_v[pl.ds(o, LANES)]
        d16 = di_v[pl.ds(o, LANES)]
        rx_v[pl.ds(o, LANES)] = (plsc.load_gather(px_v, [s16])
                                 - plsc.load_gather(px_v, [d16]))
        ry_v[pl.ds(o, LANES)] = (plsc.load_gather(py_v, [s16])
                                 - plsc.load_gather(py_v, [d16]))
        rz_v[pl.ds(o, LANES)] = (plsc.load_gather(pz_v, [s16])
                                 - plsc.load_gather(pz_v, [d16]))

    pltpu.sync_copy(rx_v, rpx_hbm.at[pl.ds(base, EW)])
    pltpu.sync_copy(ry_v, rpy_hbm.at[pl.ds(base, EW)])
    pltpu.sync_copy(rz_v, rpz_hbm.at[pl.ds(base, EW)])


def _sc_relpos(posx, posy, posz, srcp, dstp):
    f32 = jnp.float32
    kern = pl.kernel(
        _sca_body,
        out_type=[jax.ShapeDtypeStruct((EPAD,), f32)] * 3,
        mesh=_sc_mesh(),
        compiler_params=_sc_params(),
        scratch_types=[pltpu.VMEM((N,), f32)] * 3
        + [pltpu.VMEM((EW,), jnp.int32)] * 2
        + [pltpu.VMEM((EW,), f32)] * 3
        + [pltpu.SemaphoreType.DMA],
    )
    return kern(posx, posy, posz, srcp, dstp)


GB = 128
EPB = EPAD // NS


def _scb_body(q0_hbm, k0_hbm, q1_hbm, k1_hbm, src_hbm, dst_hbm,
              p0_hbm, p1_hbm,
              sia_v, dia_v, qb0_v, kb0_v, qb1_v, kb1_v, pba_v,
              sq0, sk0, sq1, sk1):
    cid = lax.axis_index("c")
    sid = lax.axis_index("s")
    base = sid * EPB
    lanes = lax.iota(jnp.int32, LANES)
    perms = [(lanes + k) & (LANES - 1) for k in (8, 4, 2, 1)]
    lane0 = lanes == 0

    def do_half(q_hbm, k_hbm, p_hbm):
        pltpu.sync_copy(src_hbm.at[pl.ds(base, EPB)], sia_v)
        pltpu.sync_copy(dst_hbm.at[pl.ds(base, EPB)], dia_v)

        def fire(b, qb, kb, sq, sk):
            pltpu.async_copy(q_hbm.at[dia_v.at[pl.ds(b, GB)]], qb, sq)
            pltpu.async_copy(k_hbm.at[sia_v.at[pl.ds(b, GB)]], kb, sk)

        def wait(qb, kb, sq, sk):
            pltpu.make_async_copy(q_hbm.at[pl.ds(0, GB)], qb, sq).wait()
            pltpu.make_async_copy(k_hbm.at[pl.ds(0, GB)], kb, sk).wait()

        def compute(b, qb, kb):
            @pl.loop(0, GB, step=LANES)
            def _(g):
                for e in range(LANES):
                    row = g + e
                    acc0 = qb[row, pl.ds(0, LANES)] * kb[row, pl.ds(0, LANES)]
                    acc1 = (qb[row, pl.ds(LANES, LANES)]
                            * kb[row, pl.ds(LANES, LANES)])
                    for c in range(2, 8, 2):
                        acc0 += (qb[row, pl.ds(c * LANES, LANES)]
                                 * kb[row, pl.ds(c * LANES, LANES)])
                        acc1 += (qb[row, pl.ds((c + 1) * LANES, LANES)]
                                 * kb[row, pl.ds((c + 1) * LANES, LANES)])
                    r = acc0 + acc1
                    for p in perms:
                        r = r + _permute(r, p)
                    plsc.store_scatter(
                        pba_v, [jnp.full((LANES,), b + row, jnp.int32)], r,
                        mask=lane0)

        fire(0, qb0_v, kb0_v, sq0, sk0)
        fire(GB, qb1_v, kb1_v, sq1, sk1)

        @pl.loop(0, EPB, step=2 * GB)
        def _(b):
            wait(qb0_v, kb0_v, sq0, sk0)
            compute(b, qb0_v, kb0_v)

            @pl.when(b + 2 * GB < EPB)
            def _():
                fire(b + 2 * GB, qb0_v, kb0_v, sq0, sk0)

            wait(qb1_v, kb1_v, sq1, sk1)
            compute(b + GB, qb1_v, kb1_v)

            @pl.when(b + 3 * GB < EPB)
            def _():
                fire(b + 3 * GB, qb1_v, kb1_v, sq1, sk1)

        pltpu.sync_copy(pba_v, p_hbm.at[pl.ds(base, EPB)])

    @pl.when(cid == 0)
    def _():
        do_half(q0_hbm, k0_hbm, p0_hbm)

    @pl.when(cid == 1)
    def _():
        do_half(q1_hbm, k1_hbm, p1_hbm)


def _sc_partial_dots(q0, k0, q1, k1, srcp, dstp):
    f32 = jnp.float32
    kern = pl.kernel(
        _scb_body,
        out_type=[jax.ShapeDtypeStruct((EPAD,), f32)] * 2,
        mesh=_sc_mesh(),
        compiler_params=_sc_params(),
        scratch_types=[pltpu.VMEM((EPB,), jnp.int32)] * 2
        + [pltpu.VMEM((GB, 128), f32)] * 4
        + [pltpu.VMEM((EPB,), f32)]
        + [pltpu.SemaphoreType.DMA] * 4,
    )
    return kern(q0, k0, q1, k1, srcp, dstp)


EB = 640
NEB = EPAD // EB


def _posmlp_body(rpx_ref, rpy_ref, rpz_ref, cols_ref, wp2_ref, bp2_ref,
                 pe0_ref, pe1_ref):
    rx = rpx_ref[0]
    ry = rpy_ref[0]
    rz = rpz_ref[0]
    cols = cols_ref[...]
    p = jnp.maximum(cols[:, 0:1] * rx + cols[:, 1:2] * ry
                    + cols[:, 2:3] * rz + cols[:, 3:4], 0.0)
    pe = lax.dot_general(p, wp2_ref[...], (((0,), (0,)), ((), ())),
                         preferred_element_type=jnp.float32) + bp2_ref[...]
    pe0_ref[...] = pe[:, :128]
    pe1_ref[...] = pe[:, 128:]


def _tc_pos_mlp(rpx, rpy, rpz, Wp1, bp1, Wp2, bp2):
    f32 = jnp.float32
    rpx3 = rpx.reshape(NEB, 1, EB)
    rpy3 = rpy.reshape(NEB, 1, EB)
    rpz3 = rpz.reshape(NEB, 1, EB)
    cols = jnp.concatenate([Wp1.T, bp1[:, None]], axis=1)
    r_spec = pl.BlockSpec((1, 1, EB), lambda i: (i, 0, 0))
    out = pl.pallas_call(
        _posmlp_body,
        grid=(NEB,),
        in_specs=[r_spec, r_spec, r_spec,
                  pl.BlockSpec((D, 4), lambda i: (0, 0)),
                  pl.BlockSpec((D, D), lambda i: (0, 0)),
                  pl.BlockSpec((1, D), lambda i: (0, 0))],
        out_specs=[pl.BlockSpec((EB, 128), lambda i: (i, 0))] * 2,
        out_shape=[jax.ShapeDtypeStruct((EPAD, 128), f32)] * 2,
    )(rpx3, rpy3, rpz3, cols, Wp2, bp2.reshape(1, D))
    return out


GBC = 64
ROWS_PER_TILE = NPAD // NS
NBT = EPB // GBC


def _scc_body(v0_hbm, v1_hbm, pe0_hbm, pe1_hbm, p0_hbm, p1_hbm,
              src_hbm, dst2_hbm, zrows_hbm, zvec_hbm,
              s0_hbm, s1_hbm, den_hbm,
              sia_v, di2_v, p00_v, p10_v, p01_v, p11_v,
              vb0_v, vb1_v, pt0_v, pt1_v, wb_v,
              acc_sh, dacc_sh, sv0, sp0, sv1, sp1):
    cid = lax.axis_index("c")
    sid = lax.axis_index("s")
    base = sid * EPB
    zbase = sid * ROWS_PER_TILE
    lanes = lax.iota(jnp.int32, LANES)
    chunk_rows = [lanes + c * LANES for c in range(8)]
    inv_scale = jnp.float32(1.0 / SCALE)

    pltpu.sync_copy(zrows_hbm.at[pl.ds(zbase, ROWS_PER_TILE)],
                    acc_sh.at[pl.ds(zbase, ROWS_PER_TILE)])

    @pl.when(cid == 0)
    def _():
        pltpu.sync_copy(zvec_hbm.at[pl.ds(zbase, ROWS_PER_TILE)],
                        dacc_sh.at[pl.ds(zbase, ROWS_PER_TILE)])

    plsc.subcore_barrier()

    def do_half(v_hbm, pe_hbm):
        pltpu.sync_copy(src_hbm.at[pl.ds(base, EPB)], sia_v)

        def fire(bi, vb, pt, p0b, p1b, sv, sp):
            pltpu.async_copy(v_hbm.at[sia_v.at[pl.ds(bi * GBC, GBC)]], vb, sv)
            pltpu.async_copy(pe_hbm.at[pl.ds(base + bi * GBC, GBC)], pt, sp)
            pltpu.async_copy(p0_hbm.at[pl.ds(base + bi * GBC, GBC)], p0b, sp)
            pltpu.async_copy(p1_hbm.at[pl.ds(base + bi * GBC, GBC)], p1b, sp)

        def wait(vb, pt, p0b, p1b, sv, sp):
            pltpu.make_async_copy(v_hbm.at[pl.ds(0, GBC)], vb, sv).wait()
            pltpu.make_async_copy(pe_hbm.at[pl.ds(0, GBC)], pt, sp).wait()
            pltpu.make_async_copy(p0_hbm.at[pl.ds(0, GBC)], p0b, sp).wait()
            pltpu.make_async_copy(p1_hbm.at[pl.ds(0, GBC)], p1b, sp).wait()

        def compute(bi, vb, pt, p0b, p1b):
            eb = base + bi * GBC

            @pl.loop(0, GBC, step=LANES)
            def _(g):
                w16 = jnp.exp((p0b[pl.ds(g, LANES)]
                               + p1b[pl.ds(g, LANES)]) * inv_scale)
                eid = eb + g + lanes
                w16 = jnp.where(eid < E, w16, jnp.float32(0.0))
                wb_v[pl.ds(g, LANES)] = w16
                for e in range(LANES):
                    row = g + e
                    rowsplat = jnp.full((LANES,), row, jnp.int32)
                    wvec = plsc.load_gather(wb_v, [rowsplat])
                    for c in range(8):
                        sl = pl.ds(c * LANES, LANES)
                        vb[row, sl] = (vb[row, sl] + pt[row, sl]) * wvec

        def scatter(bi, par, vb):
            pltpu.sync_copy(dst2_hbm.at[pl.ds(sid * NBT + bi, 1)],
                            di2_v.at[pl.ds(par, 1)])
            di_row = di2_v.at[par]
            pltpu.sync_copy(vb, acc_sh.at[di_row], add=True)

            @pl.when(cid == 0)
            def _():
                pltpu.sync_copy(wb_v, dacc_sh.at[di_row], add=True)

        fire(0, vb0_v, pt0_v, p00_v, p10_v, sv0, sp0)
        fire(1, vb1_v, pt1_v, p01_v, p11_v, sv1, sp1)

        @pl.loop(0, NBT, step=2)
        def _(bi):
            wait(vb0_v, pt0_v, p00_v, p10_v, sv0, sp0)
            compute(bi, vb0_v, pt0_v, p00_v, p10_v)
            scatter(bi, 0, vb0_v)

            @pl.when(bi + 2 < NBT)
            def _():
                fire(bi + 2, vb0_v, pt0_v, p00_v, p10_v, sv0, sp0)

            wait(vb1_v, pt1_v, p01_v, p11_v, sv1, sp1)
            compute(bi + 1, vb1_v, pt1_v, p01_v, p11_v)
            scatter(bi + 1, 1, vb1_v)

            @pl.when(bi + 3 < NBT)
            def _():
                fire(bi + 3, vb1_v, pt1_v, p01_v, p11_v, sv1, sp1)

    @pl.when(cid == 0)
    def _():
        do_half(v0_hbm, pe0_hbm)

    @pl.when(cid == 1)
    def _():
        do_half(v1_hbm, pe1_hbm)

    plsc.subcore_barrier()

    @pl.when(cid == 0)
    def _():
        pltpu.sync_copy(acc_sh.at[pl.ds(zbase, ROWS_PER_TILE)],
                        s0_hbm.at[pl.ds(zbase, ROWS_PER_TILE)])
        pltpu.sync_copy(dacc_sh.at[pl.ds(zbase, ROWS_PER_TILE)],
                        den_hbm.at[pl.ds(zbase, ROWS_PER_TILE)])

    @pl.when(cid == 1)
    def _():
        pltpu.sync_copy(acc_sh.at[pl.ds(zbase, ROWS_PER_TILE)],
                        s1_hbm.at[pl.ds(zbase, ROWS_PER_TILE)])


def _sc_aggregate(v0, v1, pe0, pe1, p0, p1, srcp, dstp):
    f32 = jnp.float32
    zrows = jnp.zeros((NPAD, 128), f32)
    zvec = jnp.zeros((NPAD,), f32)
    dst2 = dstp.reshape(EPAD // GBC, GBC)
    kern = pl.kernel(
        _scc_body,
        out_type=[jax.ShapeDtypeStruct((NPAD, 128), f32)] * 2
        + [jax.ShapeDtypeStruct((NPAD,), f32)],
        mesh=_sc_mesh(),
        compiler_params=_sc_params(),
        scratch_types=[pltpu.VMEM((EPB,), jnp.int32),
                       pltpu.VMEM((2, GBC), jnp.int32)]
        + [pltpu.VMEM((GBC,), f32)] * 4
        + [pltpu.VMEM((GBC, 128), f32)] * 4
        + [pltpu.VMEM((GBC,), f32)]
        + [pltpu.VMEM_SHARED((NPAD, 128), f32),
           pltpu.VMEM_SHARED((NPAD,), f32)]
        + [pltpu.SemaphoreType.DMA] * 4,
    )
    return kern(v0, v1, pe0, pe1, p0, p1, srcp, dst2, zrows, zvec)


def _dense_body(x_ref, w1, b1, w2, b2, wq, bq, wk, bk, wv, bv,
                h_ref, q0_ref, q1_ref, k0_ref, k1_ref, v0_ref, v1_ref):
    x = x_ref[...]
    h1 = jnp.maximum(jnp.dot(x, w1[...], preferred_element_type=jnp.float32)
                     + b1[...], 0.0)
    h = jnp.dot(h1, w2[...], preferred_element_type=jnp.float32) + b2[...]
    h_ref[...] = h
    q = jnp.dot(h, wq[...], preferred_element_type=jnp.float32) + bq[...]
    k = jnp.dot(h, wk[...], preferred_element_type=jnp.float32) + bk[...]
    v = jnp.dot(h, wv[...], preferred_element_type=jnp.float32) + bv[...]
    q0_ref[...] = q[:, :128]
    q1_ref[...] = q[:, 128:]
    k0_ref[...] = k[:, :128]
    k1_ref[...] = k[:, 128:]
    v0_ref[...] = v[:, :128]
    v1_ref[...] = v[:, 128:]


def _dense_qkv(xp, W1, b1, W2, b2, Wq, bq, Wk, bk, Wv, bv):
    grid = (NPAD // ROWB,)
    row_spec = pl.BlockSpec((ROWB, D), lambda i: (i, 0))
    half_spec = pl.BlockSpec((ROWB, 128), lambda i: (i, 0))
    w_spec = pl.BlockSpec((D, D), lambda i: (0, 0))
    b_spec = pl.BlockSpec((1, D), lambda i: (0, 0))
    out = pl.pallas_call(
        _dense_body,
        grid=grid,
        in_specs=[row_spec] + [w_spec, b_spec] * 5,
        out_specs=[row_spec] + [half_spec] * 6,
        out_shape=[jax.ShapeDtypeStruct((NPAD, D), jnp.float32)]
        + [jax.ShapeDtypeStruct((NPAD, 128), jnp.float32)] * 6,
    )(xp, W1, b1.reshape(1, D), W2, b2.reshape(1, D),
      Wq, bq.reshape(1, D), Wk, bk.reshape(1, D), Wv, bv.reshape(1, D))
    return out


def _finish_body(s0_ref, s1_ref, den_ref, h_ref, x_ref, g1, be1, g2, be2, o_ref):
    s = jnp.concatenate([s0_ref[...], s1_ref[...]], axis=-1)
    den = den_ref[...]
    h = h_ref[...]
    x = x_ref[...]
    out = s / (den + 1e-16)
    a = out + h
    mu = jnp.mean(a, axis=-1, keepdims=True)
    var = jnp.mean((a - mu) ** 2, axis=-1, keepdims=True)
    h_attn = (a - mu) * jax.lax.rsqrt(var + 1e-5) * g1[...] + be1[...]
    b = h_attn + x
    mu2 = jnp.mean(b, axis=-1, keepdims=True)
    var2 = jnp.mean((b - mu2) ** 2, axis=-1, keepdims=True)
    o_ref[...] = (b - mu2) * jax.lax.rsqrt(var2 + 1e-5) * g2[...] + be2[...]


def _finish(s0, s1, den, hp, xp, g1, be1, g2, be2):
    grid = (NPAD // ROWB,)
    row_spec = pl.BlockSpec((ROWB, D), lambda i: (i, 0))
    half_spec = pl.BlockSpec((ROWB, 128), lambda i: (i, 0))
    den_spec = pl.BlockSpec((ROWB, 1), lambda i: (i, 0))
    v_spec = pl.BlockSpec((1, D), lambda i: (0, 0))
    return pl.pallas_call(
        _finish_body,
        grid=grid,
        in_specs=[half_spec, half_spec, den_spec, row_spec, row_spec]
        + [v_spec] * 4,
        out_specs=row_spec,
        out_shape=jax.ShapeDtypeStruct((NPAD, D), jnp.float32),
    )(s0, s1, den.reshape(NPAD, 1), hp, xp,
      g1.reshape(1, D), be1.reshape(1, D), g2.reshape(1, D), be2.reshape(1, D))


def kernel(x, edge_index, pos, W1, b1, W2, b2, Wq, bq, Wk, bk, Wv, bv,
           Wp1, bp1, Wp2, bp2, g1, be1, g2, be2):
    xp = jnp.pad(x, ((0, NPAD - N), (0, 0)))
    hp, q0, q1, k0, k1, v0, v1 = _dense_qkv(
        xp, W1, b1, W2, b2, Wq, bq, Wk, bk, Wv, bv)

    src = edge_index[0]
    dst = edge_index[1]
    srcp = jnp.pad(src, (0, EPAD - E))
    dstp = jnp.pad(dst, (0, EPAD - E))
    posx = pos[:, 0]
    posy = pos[:, 1]
    posz = pos[:, 2]

    rpx, rpy, rpz = _sc_relpos(posx, posy, posz, srcp, dstp)

    p0, p1 = _sc_partial_dots(q0, k0, q1, k1, srcp, dstp)

    pe0t, pe1t = _tc_pos_mlp(rpx, rpy, rpz, Wp1, bp1, Wp2, bp2)
    s0, s1, den = _sc_aggregate(v0, v1, pe0t, pe1t, p0, p1, srcp, dstp)
    out = _finish(s0, s1, den, hp, xp, g1, be1, g2, be2)
    return out[:N]
